# Initial kernel scaffold; baseline (speedup 1.0000x reference)
import jax
import jax.numpy as jnp
from jax.experimental import pallas as pl


def kernel(boxes, confs, max_output):
    raise NotImplementedError("write your pallas kernel here")


# speedup vs baseline: 4.4079x; 4.4079x over previous
import jax
import jax.numpy as jnp
from jax.experimental import pallas as pl

_N = 20000
_NUM_CLASSES = 81
_NMS_IOU = 0.5
_MAX_OUT = 200
_SCORE_THR = 0.05
_NEG = -1e30
_LANES = 128
_ROWS = (_N + _LANES - 1) // _LANES
_NPAD = _ROWS * _LANES


def _nms_kernel(boxes_ref, confs_ref, out_ref):
    confs = confs_ref[...]
    scores = jnp.max(confs, axis=1)
    cls_iota = jax.lax.broadcasted_iota(jnp.int32, confs.shape, 1)
    cat = jnp.min(
        jnp.where(confs == scores[:, None, :], cls_iota, _NUM_CLASSES), axis=1
    )
    valid = jnp.logical_and(scores > _SCORE_THR, cat != 0)
    work0 = jnp.where(valid, scores, _NEG)

    x1 = boxes_ref[:, 0, :]
    y1 = boxes_ref[:, 1, :]
    x2 = boxes_ref[:, 2, :]
    y2 = boxes_ref[:, 3, :]
    max_coord = jnp.max(jnp.maximum(jnp.maximum(x1, y1), jnp.maximum(x2, y2)))
    catf = cat.astype(jnp.float32)
    off = catf * (max_coord + 1.0)
    x1o = x1 + off
    y1o = y1 + off
    x2o = x2 + off
    y2o = y2 + off
    area2 = (x2o - x1o) * (y2o - y1o)
    lin = (
        jax.lax.broadcasted_iota(jnp.int32, (_ROWS, _LANES), 0) * _LANES
        + jax.lax.broadcasted_iota(jnp.int32, (_ROWS, _LANES), 1)
    )
    lane = jax.lax.broadcasted_iota(jnp.int32, (1, _LANES), 1)

    def body(t, work):
        m = jnp.max(work)
        idx = jnp.min(jnp.where(work == m, lin, _NPAD))
        first = lin == idx
        f = first.astype(jnp.float32)
        x1s = jnp.sum(x1o * f)
        y1s = jnp.sum(y1o * f)
        x2s = jnp.sum(x2o * f)
        y2s = jnp.sum(y2o * f)
        cs = jnp.sum(catf * f)
        ltx = jnp.maximum(x1s, x1o)
        lty = jnp.maximum(y1s, y1o)
        rbx = jnp.minimum(x2s, x2o)
        rby = jnp.minimum(y2s, y2o)
        w = jnp.maximum(rbx - ltx, 0.0)
        h = jnp.maximum(rby - lty, 0.0)
        inter = w * h
        area1 = (x2s - x1s) * (y2s - y1s)
        iou = inter / (area1 + area2 - inter + 1e-9)
        new_work = jnp.where(jnp.logical_or(iou > _NMS_IOU, first), _NEG, work)
        vm = (m > _NEG / 2).astype(jnp.float32)
        offs = cs * (max_coord + 1.0)
        vals = (x1s - offs, y1s - offs, x2s - offs, y2s - offs, m, cs)
        row = jnp.zeros((1, _LANES), jnp.float32)
        for k, v in enumerate(vals):
            row = row + jnp.where(lane == k, v * vm, 0.0)
        out_ref[pl.ds(t, 1), :] = row
        return new_work

    jax.lax.fori_loop(0, _MAX_OUT, body, work0)


def _run(boxes3, confs3, interpret=False):
    return pl.pallas_call(
        _nms_kernel,
        out_shape=jax.ShapeDtypeStruct((_MAX_OUT, _LANES), jnp.float32),
        interpret=interpret,
    )(boxes3, confs3)


def kernel(boxes, confs, max_output):
    boxes_p = jnp.pad(boxes, ((0, _NPAD - _N), (0, 0)))
    confs_p = jnp.pad(confs, ((0, _NPAD - _N), (0, 0)), constant_values=-1.0)
    boxes3 = boxes_p.reshape(_ROWS, _LANES, 4).transpose(0, 2, 1)
    confs3 = confs_p.reshape(_ROWS, _LANES, _NUM_CLASSES).transpose(0, 2, 1)
    out = _run(boxes3, confs3)
    mask = jnp.arange(_MAX_OUT) < max_output
    mf = mask.astype(jnp.float32)
    boxes_out = out[:, 0:4] * mf[:, None]
    cats_out = jnp.where(mask, out[:, 5].astype(jnp.int32), 0)
    scores_out = out[:, 4] * mf
    return boxes_out, cats_out, scores_out


# speedup vs baseline: 11.8982x; 2.6993x over previous
import jax
import jax.numpy as jnp
from jax.experimental import pallas as pl
from jax.experimental.pallas import tpu as pltpu

_N = 20000
_NUM_CLASSES = 81
_NMS_IOU = 0.5
_MAX_OUT = 200
_SCORE_THR = 0.05
_NEG = -1e30
_LANES = 128
_ROWS = (_N + _LANES - 1) // _LANES
_NPAD = _ROWS * _LANES


def _nms_kernel(boxes_ref, confs_ref, out_ref,
                x1_ref, y1_ref, x2_ref, y2_ref, cat_ref, a2_ref):
    confs = confs_ref[...]
    scores = jnp.max(confs, axis=1)
    cls_iota = jax.lax.broadcasted_iota(jnp.int32, confs.shape, 1)
    cat = jnp.min(
        jnp.where(confs == scores[:, None, :], cls_iota, _NUM_CLASSES), axis=1
    )
    valid = jnp.logical_and(scores > _SCORE_THR, cat != 0)
    work0 = jnp.where(valid, scores, _NEG)

    x1 = boxes_ref[:, 0, :]
    y1 = boxes_ref[:, 1, :]
    x2 = boxes_ref[:, 2, :]
    y2 = boxes_ref[:, 3, :]
    max_coord = jnp.max(jnp.maximum(jnp.maximum(x1, y1), jnp.maximum(x2, y2)))
    catf = cat.astype(jnp.float32)
    off = catf * (max_coord + 1.0)
    x1_ref[...] = x1 + off
    y1_ref[...] = y1 + off
    x2_ref[...] = x2 + off
    y2_ref[...] = y2 + off
    cat_ref[...] = catf
    a2_ref[...] = (x2_ref[...] - x1_ref[...]) * (y2_ref[...] - y1_ref[...])
    lin = (
        jax.lax.broadcasted_iota(jnp.int32, (_ROWS, _LANES), 0) * _LANES
        + jax.lax.broadcasted_iota(jnp.int32, (_ROWS, _LANES), 1)
    )
    lane = jax.lax.broadcasted_iota(jnp.int32, (1, _LANES), 1)

    def _pick(ref, r, onehot):
        return jnp.sum(jnp.where(onehot, ref[pl.ds(r, 1), :], 0.0))

    def body(t, work):
        m = jnp.max(work)
        idx = jnp.min(jnp.where(work == m, lin, _NPAD))
        r = idx // _LANES
        onehot = lane == (idx - r * _LANES)
        x1s = _pick(x1_ref, r, onehot)
        y1s = _pick(y1_ref, r, onehot)
        x2s = _pick(x2_ref, r, onehot)
        y2s = _pick(y2_ref, r, onehot)
        cs = _pick(cat_ref, r, onehot)
        ltx = jnp.maximum(x1s, x1_ref[...])
        lty = jnp.maximum(y1s, y1_ref[...])
        rbx = jnp.minimum(x2s, x2_ref[...])
        rby = jnp.minimum(y2s, y2_ref[...])
        w = jnp.maximum(rbx - ltx, 0.0)
        h = jnp.maximum(rby - lty, 0.0)
        inter = w * h
        area1 = (x2s - x1s) * (y2s - y1s)
        iou = inter / (area1 + a2_ref[...] - inter + 1e-9)
        new_work = jnp.where(iou > _NMS_IOU, _NEG, work)
        vm = (m > _NEG / 2).astype(jnp.float32)
        offs = cs * (max_coord + 1.0)
        vals = (x1s - offs, y1s - offs, x2s - offs, y2s - offs, m, cs)
        row = jnp.zeros((1, _LANES), jnp.float32)
        for k, v in enumerate(vals):
            row = row + jnp.where(lane == k, v * vm, 0.0)
        out_ref[pl.ds(t, 1), :] = row
        return new_work

    jax.lax.fori_loop(0, _MAX_OUT, body, work0)


def _run(boxes3, confs3, interpret=False):
    return pl.pallas_call(
        _nms_kernel,
        out_shape=jax.ShapeDtypeStruct((_MAX_OUT, _LANES), jnp.float32),
        scratch_shapes=[pltpu.VMEM((_ROWS, _LANES), jnp.float32)] * 6,
        interpret=interpret,
    )(boxes3, confs3)


def kernel(boxes, confs, max_output):
    boxes_p = jnp.pad(boxes, ((0, _NPAD - _N), (0, 0)))
    confs_p = jnp.pad(confs, ((0, _NPAD - _N), (0, 0)), constant_values=-1.0)
    boxes3 = boxes_p.reshape(_ROWS, _LANES, 4).transpose(0, 2, 1)
    confs3 = confs_p.reshape(_ROWS, _LANES, _NUM_CLASSES).transpose(0, 2, 1)
    out = _run(boxes3, confs3)
    mask = jnp.arange(_MAX_OUT) < max_output
    mf = mask.astype(jnp.float32)
    boxes_out = out[:, 0:4] * mf[:, None]
    cats_out = jnp.where(mask, out[:, 5].astype(jnp.int32), 0)
    scores_out = out[:, 4] * mf
    return boxes_out, cats_out, scores_out
